# Initial kernel scaffold; baseline (speedup 1.0000x reference)
import functools

import jax
import jax.numpy as jnp
from jax import lax
from jax.experimental import pallas as pl
from jax.experimental.pallas import tpu as pltpu
from jax.experimental.pallas import tpu_sc as plsc

N_NODES = 10000
N_EDGES = 320000
DIM_H = 128
N_HEADS = 8
HEAD_DIM = 16
INV_SQRT_D = 0.25

NC = 2
NS = 16
NW = NC * NS
EDGES_PER_TILE = N_EDGES // NW
CHUNK = 80
N_CHUNKS = EDGES_PER_TILE // CHUNK
ROWS_PER_TILE = N_NODES // NS

_f32 = jnp.float32
_i32 = jnp.int32


def _iota16():
    return lax.iota(_i32, 16)


def _full16(v):
    return jnp.full((16,), v, _i32)


_QKV_BLOCK = 1000
_DN = (((1,), (1,)), ((), ()))


def _qkv_body(x_ref, wq_ref, wk_ref, wv_ref, bq_ref, bk_ref, bv_ref,
              q_ref, k_ref, v_ref):
    xx = x_ref[...]
    q_ref[...] = lax.dot_general(xx, wq_ref[...], _DN,
                                 preferred_element_type=_f32) + bq_ref[...]
    k_ref[...] = lax.dot_general(xx, wk_ref[...], _DN,
                                 preferred_element_type=_f32) + bk_ref[...]
    v_ref[...] = lax.dot_general(xx, wv_ref[...], _DN,
                                 preferred_element_type=_f32) + bv_ref[...]


def _qkv(x, WQ, WK, WV, bQ, bK, bV):
    n_blk = N_NODES // _QKV_BLOCK
    blk = pl.BlockSpec((_QKV_BLOCK, DIM_H), lambda i: (i, 0))
    wblk = pl.BlockSpec((DIM_H, DIM_H), lambda i: (0, 0))
    bblk = pl.BlockSpec((1, DIM_H), lambda i: (0, 0))
    out = jax.ShapeDtypeStruct((N_NODES, DIM_H), _f32)
    return pl.pallas_call(
        _qkv_body,
        grid=(n_blk,),
        in_specs=[blk, wblk, wblk, wblk, bblk, bblk, bblk],
        out_specs=[blk, blk, blk],
        out_shape=[out, out, out],
    )(x, WQ, WK, WV, bQ.reshape(1, DIM_H), bK.reshape(1, DIM_H),
      bV.reshape(1, DIM_H))


def _combine_body(a_ref, b_ref, o_ref):
    o_ref[...] = a_ref[...] + b_ref[...]


def _combine(a, b):
    n_blk = N_NODES // _QKV_BLOCK
    blk = pl.BlockSpec((_QKV_BLOCK, DIM_H), lambda i: (i, 0))
    return pl.pallas_call(
        _combine_body,
        grid=(n_blk,),
        in_specs=[blk, blk],
        out_specs=blk,
        out_shape=jax.ShapeDtypeStruct((N_NODES, DIM_H), _f32),
    )(a, b)


def _mesh():
    return plsc.VectorSubcoreMesh(core_axis_name="c", subcore_axis_name="s",
                                  num_cores=NC, num_subcores=NS)


def _pass_a_body(q_hbm, k_hbm, eb_hbm, src_hbm, dst_hbm, z8_hbm,
                 ex_hbm, den0_hbm, den1_hbm,
                 src_v, dst_v, qg, kg, bg, exb, den_sh):
    c = lax.axis_index("c")
    s = lax.axis_index("s")
    wid = s * NC + c
    row0 = s * ROWS_PER_TILE
    pltpu.sync_copy(z8_hbm.at[pl.ds(row0, ROWS_PER_TILE)],
                    den_sh.at[pl.ds(row0, ROWS_PER_TILE)])
    plsc.subcore_barrier()

    base0 = wid * EDGES_PER_TILE

    def chunk(it, carry):
        base = base0 + it * CHUNK
        pltpu.sync_copy(src_hbm.at[pl.ds(base, CHUNK)], src_v)
        pltpu.sync_copy(dst_hbm.at[pl.ds(base, CHUNK)], dst_v)
        pltpu.sync_copy(q_hbm.at[dst_v], qg)
        pltpu.sync_copy(k_hbm.at[src_v], kg)
        pltpu.sync_copy(eb_hbm.at[pl.ds(base, CHUNK)], bg)
        for g in range(CHUNK // 16):
            row = g * 16 + _iota16()
            for h in range(N_HEADS):
                acc = jnp.zeros((16,), _f32)
                for d in range(HEAD_DIM):
                    col = _full16(h * HEAD_DIM + d)
                    acc = acc + (plsc.load_gather(qg, [row, col]) *
                                 plsc.load_gather(kg, [row, col]))
                bias = plsc.load_gather(bg, [row, _full16(h)])
                ex = jnp.exp(acc * INV_SQRT_D + bias)
                plsc.store_scatter(exb, [row, _full16(h)], ex)
        pltpu.sync_copy(exb, ex_hbm.at[pl.ds(base, CHUNK)])
        pltpu.sync_copy(exb, den_sh.at[dst_v], add=True)
        return carry

    lax.fori_loop(0, N_CHUNKS, chunk, 0)
    plsc.subcore_barrier()

    @pl.when(c == 0)
    def _():
        pltpu.sync_copy(den_sh.at[pl.ds(row0, ROWS_PER_TILE)],
                        den0_hbm.at[pl.ds(row0, ROWS_PER_TILE)])

    @pl.when(c == 1)
    def _():
        pltpu.sync_copy(den_sh.at[pl.ds(row0, ROWS_PER_TILE)],
                        den1_hbm.at[pl.ds(row0, ROWS_PER_TILE)])


def _pass_a(Q, K, edge_bias, src, dst, z8):
    out = (jax.ShapeDtypeStruct((N_EDGES, N_HEADS), _f32),
           jax.ShapeDtypeStruct((N_NODES, N_HEADS), _f32),
           jax.ShapeDtypeStruct((N_NODES, N_HEADS), _f32))
    k = pl.kernel(
        _pass_a_body,
        out_type=out,
        mesh=_mesh(),
        scratch_types=[
            pltpu.VMEM((CHUNK,), _i32),
            pltpu.VMEM((CHUNK,), _i32),
            pltpu.VMEM((CHUNK, DIM_H), _f32),
            pltpu.VMEM((CHUNK, DIM_H), _f32),
            pltpu.VMEM((CHUNK, N_HEADS), _f32),
            pltpu.VMEM((CHUNK, N_HEADS), _f32),
            pltpu.VMEM_SHARED((N_NODES, N_HEADS), _f32),
        ],
    )
    return k(Q, K, edge_bias, src, dst, z8)


def _pass_b_body(v_hbm, ex_hbm, den0_hbm, den1_hbm, src_hbm, dst_hbm, z128_hbm,
                 alpha_hbm, agg0_hbm, agg1_hbm,
                 src_v, dst_v, vg, exb, d0g, d1g, ab, agg_sh):
    c = lax.axis_index("c")
    s = lax.axis_index("s")
    wid = s * NC + c
    row0 = s * ROWS_PER_TILE
    pltpu.sync_copy(z128_hbm.at[pl.ds(row0, ROWS_PER_TILE)],
                    agg_sh.at[pl.ds(row0, ROWS_PER_TILE)])
    plsc.subcore_barrier()

    base0 = wid * EDGES_PER_TILE

    def chunk(it, carry):
        base = base0 + it * CHUNK
        pltpu.sync_copy(src_hbm.at[pl.ds(base, CHUNK)], src_v)
        pltpu.sync_copy(dst_hbm.at[pl.ds(base, CHUNK)], dst_v)
        pltpu.sync_copy(ex_hbm.at[pl.ds(base, CHUNK)], exb)
        pltpu.sync_copy(den0_hbm.at[dst_v], d0g)
        pltpu.sync_copy(den1_hbm.at[dst_v], d1g)
        pltpu.sync_copy(v_hbm.at[src_v], vg)
        for g in range(CHUNK // 16):
            row = g * 16 + _iota16()
            for h in range(N_HEADS):
                fh = _full16(h)
                ex = plsc.load_gather(exb, [row, fh])
                den = (plsc.load_gather(d0g, [row, fh]) +
                       plsc.load_gather(d1g, [row, fh]))
                al = ex / (den + 1e-16)
                plsc.store_scatter(ab, [row, fh], al)
                for d in range(HEAD_DIM):
                    col = _full16(h * HEAD_DIM + d)
                    vv = plsc.load_gather(vg, [row, col])
                    plsc.store_scatter(vg, [row, col], vv * al)
        pltpu.sync_copy(ab, alpha_hbm.at[pl.ds(base, CHUNK)])
        pltpu.sync_copy(vg, agg_sh.at[dst_v], add=True)
        return carry

    lax.fori_loop(0, N_CHUNKS, chunk, 0)
    plsc.subcore_barrier()

    @pl.when(c == 0)
    def _():
        pltpu.sync_copy(agg_sh.at[pl.ds(row0, ROWS_PER_TILE)],
                        agg0_hbm.at[pl.ds(row0, ROWS_PER_TILE)])

    @pl.when(c == 1)
    def _():
        pltpu.sync_copy(agg_sh.at[pl.ds(row0, ROWS_PER_TILE)],
                        agg1_hbm.at[pl.ds(row0, ROWS_PER_TILE)])


def _pass_b(V, ex, den0, den1, src, dst, z128):
    out = (jax.ShapeDtypeStruct((N_EDGES, N_HEADS), _f32),
           jax.ShapeDtypeStruct((N_NODES, DIM_H), _f32),
           jax.ShapeDtypeStruct((N_NODES, DIM_H), _f32))
    k = pl.kernel(
        _pass_b_body,
        out_type=out,
        mesh=_mesh(),
        scratch_types=[
            pltpu.VMEM((CHUNK,), _i32),
            pltpu.VMEM((CHUNK,), _i32),
            pltpu.VMEM((CHUNK, DIM_H), _f32),
            pltpu.VMEM((CHUNK, N_HEADS), _f32),
            pltpu.VMEM((CHUNK, N_HEADS), _f32),
            pltpu.VMEM((CHUNK, N_HEADS), _f32),
            pltpu.VMEM((CHUNK, N_HEADS), _f32),
            pltpu.VMEM_SHARED((N_NODES, DIM_H), _f32),
        ],
    )
    return k(V, ex, den0, den1, src, dst, z128)


def kernel(x, edge_index, edge_bias, WQ, bQ, WK, bK, WV, bV):
    src = edge_index[0]
    dst = edge_index[1]
    z8 = jnp.zeros((N_NODES, N_HEADS), _f32)
    z128 = jnp.zeros((N_NODES, DIM_H), _f32)
    Q, K, V = _qkv(x, WQ, WK, WV, bQ, bK, bV)
    ex, den0, den1 = _pass_a(Q, K, edge_bias, src, dst, z8)
    alpha, agg0, agg1 = _pass_b(V, ex, den0, den1, src, dst, z128)
    h = _combine(agg0, agg1)
    return (h, alpha)


# speedup vs baseline: 10.2170x; 10.2170x over previous
import functools

import jax
import jax.numpy as jnp
from jax import lax
from jax.experimental import pallas as pl
from jax.experimental.pallas import tpu as pltpu
from jax.experimental.pallas import tpu_sc as plsc

N_NODES = 10000
N_EDGES = 320000
DIM_H = 128
N_HEADS = 8
HEAD_DIM = 16
INV_SQRT_D = 0.25

NC = 2
NS = 16
NW = NC * NS
EDGES_PER_TILE = N_EDGES // NW
CHUNK = 80
N_CHUNKS = EDGES_PER_TILE // CHUNK
ROW_STRIDE = 624
ROW_WIN = 640

_f32 = jnp.float32
_i32 = jnp.int32


def _iota16():
    return lax.iota(_i32, 16)


def _full16(v):
    return jnp.full((16,), v, _i32)


_QKV_BLOCK = 1000
_DN = (((1,), (1,)), ((), ()))


def _qkv_body(x_ref, wq_ref, wk_ref, wv_ref, bq_ref, bk_ref, bv_ref,
              q_ref, k_ref, v_ref):
    xx = x_ref[...]
    q_ref[...] = lax.dot_general(xx, wq_ref[...], _DN,
                                 preferred_element_type=_f32) + bq_ref[...]
    k_ref[...] = lax.dot_general(xx, wk_ref[...], _DN,
                                 preferred_element_type=_f32) + bk_ref[...]
    v_ref[...] = lax.dot_general(xx, wv_ref[...], _DN,
                                 preferred_element_type=_f32) + bv_ref[...]


def _qkv(x, WQ, WK, WV, bQ, bK, bV):
    n_blk = N_NODES // _QKV_BLOCK
    blk = pl.BlockSpec((_QKV_BLOCK, DIM_H), lambda i: (i, 0))
    wblk = pl.BlockSpec((DIM_H, DIM_H), lambda i: (0, 0))
    bblk = pl.BlockSpec((1, DIM_H), lambda i: (0, 0))
    out = jax.ShapeDtypeStruct((N_NODES, DIM_H), _f32)
    return pl.pallas_call(
        _qkv_body,
        grid=(n_blk,),
        in_specs=[blk, wblk, wblk, wblk, bblk, bblk, bblk],
        out_specs=[blk, blk, blk],
        out_shape=[out, out, out],
    )(x, WQ, WK, WV, bQ.reshape(1, DIM_H), bK.reshape(1, DIM_H),
      bV.reshape(1, DIM_H))


def _combine_body(a_ref, b_ref, o_ref):
    o_ref[...] = a_ref[...] + b_ref[...]


def _combine(a, b):
    n_blk = N_NODES // _QKV_BLOCK
    blk = pl.BlockSpec((_QKV_BLOCK, DIM_H), lambda i: (i, 0))
    return pl.pallas_call(
        _combine_body,
        grid=(n_blk,),
        in_specs=[blk, blk],
        out_specs=blk,
        out_shape=jax.ShapeDtypeStruct((N_NODES, DIM_H), _f32),
    )(a, b)


def _mesh():
    return plsc.VectorSubcoreMesh(core_axis_name="c", subcore_axis_name="s",
                                  num_cores=NC, num_subcores=NS)


_SC_PARAMS = pltpu.CompilerParams(needs_layout_passes=False,
                                  use_tc_tiling_on_sc=False)


def _pass_a_body(q_hbm, k_hbm, eb_hbm, src_hbm, dst_hbm, z8_hbm,
                 ex_hbm, den0_hbm, den1_hbm,
                 src_v, dst_v, qg, kg, bg, exb, den_sh):
    c = lax.axis_index("c")
    s = lax.axis_index("s")
    wid = s * NC + c
    row0 = s * ROW_STRIDE
    pltpu.sync_copy(z8_hbm.at[pl.ds(row0, ROW_WIN)],
                    den_sh.at[pl.ds(row0, ROW_WIN)])
    plsc.subcore_barrier()

    base0 = wid * EDGES_PER_TILE

    def chunk(it, carry):
        base = base0 + it * CHUNK
        pltpu.sync_copy(src_hbm.at[pl.ds(base, CHUNK)], src_v)
        pltpu.sync_copy(dst_hbm.at[pl.ds(base, CHUNK)], dst_v)
        pltpu.sync_copy(q_hbm.at[dst_v], qg)
        pltpu.sync_copy(k_hbm.at[src_v], kg)
        pltpu.sync_copy(eb_hbm.at[pl.ds(base, CHUNK)], bg)
        for g in range(CHUNK // 16):
            row = g * 16 + _iota16()
            for h in range(N_HEADS):
                acc = jnp.zeros((16,), _f32)
                for d in range(HEAD_DIM):
                    col = _full16(h * HEAD_DIM + d)
                    acc = acc + (plsc.load_gather(qg, [row, col]) *
                                 plsc.load_gather(kg, [row, col]))
                bias = plsc.load_gather(bg, [row, _full16(h)])
                ex = jnp.exp(acc * INV_SQRT_D + bias)
                plsc.store_scatter(exb, [row, _full16(h)], ex)
        pltpu.sync_copy(exb, ex_hbm.at[pl.ds(base, CHUNK)])
        pltpu.sync_copy(exb, den_sh.at[dst_v], add=True)
        return carry

    lax.fori_loop(0, N_CHUNKS, chunk, 0)
    plsc.subcore_barrier()

    @pl.when(c == 0)
    def _():
        pltpu.sync_copy(den_sh.at[pl.ds(row0, ROW_WIN)],
                        den0_hbm.at[pl.ds(row0, ROW_WIN)])

    @pl.when(c == 1)
    def _():
        pltpu.sync_copy(den_sh.at[pl.ds(row0, ROW_WIN)],
                        den1_hbm.at[pl.ds(row0, ROW_WIN)])


def _pass_a(Q, K, edge_bias, src, dst, z8):
    out = (jax.ShapeDtypeStruct((N_EDGES, N_HEADS), _f32),
           jax.ShapeDtypeStruct((N_NODES, N_HEADS), _f32),
           jax.ShapeDtypeStruct((N_NODES, N_HEADS), _f32))
    k = pl.kernel(
        _pass_a_body,
        out_type=out,
        mesh=_mesh(),
        compiler_params=_SC_PARAMS,
        scratch_types=[
            pltpu.VMEM((CHUNK,), _i32),
            pltpu.VMEM((CHUNK,), _i32),
            pltpu.VMEM((CHUNK, DIM_H), _f32),
            pltpu.VMEM((CHUNK, DIM_H), _f32),
            pltpu.VMEM((CHUNK, N_HEADS), _f32),
            pltpu.VMEM((CHUNK, N_HEADS), _f32),
            pltpu.VMEM_SHARED((N_NODES, N_HEADS), _f32),
        ],
    )
    return k(Q, K, edge_bias, src, dst, z8)


def _pass_b_body(v_hbm, ex_hbm, den0_hbm, den1_hbm, src_hbm, dst_hbm, z128_hbm,
                 alpha_hbm, agg0_hbm, agg1_hbm,
                 src_v, dst_v, vg, exb, d0g, d1g, ab, agg_sh):
    c = lax.axis_index("c")
    s = lax.axis_index("s")
    wid = s * NC + c
    row0 = s * ROW_STRIDE
    pltpu.sync_copy(z128_hbm.at[pl.ds(row0, ROW_WIN)],
                    agg_sh.at[pl.ds(row0, ROW_WIN)])
    plsc.subcore_barrier()

    base0 = wid * EDGES_PER_TILE

    def chunk(it, carry):
        base = base0 + it * CHUNK
        pltpu.sync_copy(src_hbm.at[pl.ds(base, CHUNK)], src_v)
        pltpu.sync_copy(dst_hbm.at[pl.ds(base, CHUNK)], dst_v)
        pltpu.sync_copy(ex_hbm.at[pl.ds(base, CHUNK)], exb)
        pltpu.sync_copy(den0_hbm.at[dst_v], d0g)
        pltpu.sync_copy(den1_hbm.at[dst_v], d1g)
        pltpu.sync_copy(v_hbm.at[src_v], vg)
        for g in range(CHUNK // 16):
            row = g * 16 + _iota16()
            for h in range(N_HEADS):
                fh = _full16(h)
                ex = plsc.load_gather(exb, [row, fh])
                den = (plsc.load_gather(d0g, [row, fh]) +
                       plsc.load_gather(d1g, [row, fh]))
                al = ex / (den + 1e-16)
                plsc.store_scatter(ab, [row, fh], al)
                for d in range(HEAD_DIM):
                    col = _full16(h * HEAD_DIM + d)
                    vv = plsc.load_gather(vg, [row, col])
                    plsc.store_scatter(vg, [row, col], vv * al)
        pltpu.sync_copy(ab, alpha_hbm.at[pl.ds(base, CHUNK)])
        pltpu.sync_copy(vg, agg_sh.at[dst_v], add=True)
        return carry

    lax.fori_loop(0, N_CHUNKS, chunk, 0)
    plsc.subcore_barrier()

    @pl.when(c == 0)
    def _():
        pltpu.sync_copy(agg_sh.at[pl.ds(row0, ROW_WIN)],
                        agg0_hbm.at[pl.ds(row0, ROW_WIN)])

    @pl.when(c == 1)
    def _():
        pltpu.sync_copy(agg_sh.at[pl.ds(row0, ROW_WIN)],
                        agg1_hbm.at[pl.ds(row0, ROW_WIN)])


def _pass_b(V, ex, den0, den1, src, dst, z128):
    out = (jax.ShapeDtypeStruct((N_EDGES, N_HEADS), _f32),
           jax.ShapeDtypeStruct((N_NODES, DIM_H), _f32),
           jax.ShapeDtypeStruct((N_NODES, DIM_H), _f32))
    k = pl.kernel(
        _pass_b_body,
        out_type=out,
        mesh=_mesh(),
        compiler_params=_SC_PARAMS,
        scratch_types=[
            pltpu.VMEM((CHUNK,), _i32),
            pltpu.VMEM((CHUNK,), _i32),
            pltpu.VMEM((CHUNK, DIM_H), _f32),
            pltpu.VMEM((CHUNK, N_HEADS), _f32),
            pltpu.VMEM((CHUNK, N_HEADS), _f32),
            pltpu.VMEM((CHUNK, N_HEADS), _f32),
            pltpu.VMEM((CHUNK, N_HEADS), _f32),
            pltpu.VMEM_SHARED((N_NODES, DIM_H), _f32),
        ],
    )
    return k(V, ex, den0, den1, src, dst, z128)


def kernel(x, edge_index, edge_bias, WQ, bQ, WK, bK, WV, bV):
    src = edge_index[0]
    dst = edge_index[1]
    z8 = jnp.zeros((N_NODES, N_HEADS), _f32)
    z128 = jnp.zeros((N_NODES, DIM_H), _f32)
    Q, K, V = _qkv(x, WQ, WK, WV, bQ, bK, bV)
    ex, den0, den1 = _pass_a(Q, K, edge_bias, src, dst, z8)
    alpha, agg0, agg1 = _pass_b(V, ex, den0, den1, src, dst, z128)
    h = _combine(agg0, agg1)
    return (h, alpha)


# speedup vs baseline: 12.2156x; 1.1956x over previous
import jax
import jax.numpy as jnp
from jax import lax
from jax.experimental import pallas as pl
from jax.experimental.pallas import tpu as pltpu
from jax.experimental.pallas import tpu_sc as plsc

N_NODES = 10000
N_EDGES = 320000
DIM_H = 128
N_HEADS = 8
HEAD_DIM = 16
INV_SQRT_D = 0.25

NC = 2
NS = 16
NW = NC * NS
EDGES_PER_TILE = N_EDGES // NW
CHUNK = 80
N_CHUNKS = EDGES_PER_TILE // CHUNK
ROW_STRIDE = 624
ROW_WIN = 640

_f32 = jnp.float32
_i32 = jnp.int32


def _iota16():
    return lax.iota(_i32, 16)


def _full16(v):
    return jnp.full((16,), v, _i32)


_QKV_BLOCK = 1000
_DN = (((1,), (1,)), ((), ()))


def _qkv_body(x_ref, wq_ref, wk_ref, wv_ref, bq_ref, bk_ref, bv_ref,
              q_ref, k_ref, v_ref):
    xx = x_ref[...]
    q_ref[...] = lax.dot_general(xx, wq_ref[...], _DN,
                                 preferred_element_type=_f32) + bq_ref[...]
    k_ref[...] = lax.dot_general(xx, wk_ref[...], _DN,
                                 preferred_element_type=_f32) + bk_ref[...]
    v_ref[...] = lax.dot_general(xx, wv_ref[...], _DN,
                                 preferred_element_type=_f32) + bv_ref[...]


def _qkv(x, WQ, WK, WV, bQ, bK, bV):
    n_blk = N_NODES // _QKV_BLOCK
    blk = pl.BlockSpec((_QKV_BLOCK, DIM_H), lambda i: (i, 0))
    wblk = pl.BlockSpec((DIM_H, DIM_H), lambda i: (0, 0))
    bblk = pl.BlockSpec((1, DIM_H), lambda i: (0, 0))
    out = jax.ShapeDtypeStruct((N_NODES, DIM_H), _f32)
    return pl.pallas_call(
        _qkv_body,
        grid=(n_blk,),
        in_specs=[blk, wblk, wblk, wblk, bblk, bblk, bblk],
        out_specs=[blk, blk, blk],
        out_shape=[out, out, out],
    )(x, WQ, WK, WV, bQ.reshape(1, DIM_H), bK.reshape(1, DIM_H),
      bV.reshape(1, DIM_H))


def _combine_body(a_ref, b_ref, o_ref):
    o_ref[...] = a_ref[...] + b_ref[...]


def _combine(a, b):
    n_blk = N_NODES // _QKV_BLOCK
    blk = pl.BlockSpec((_QKV_BLOCK, DIM_H), lambda i: (i, 0))
    return pl.pallas_call(
        _combine_body,
        grid=(n_blk,),
        in_specs=[blk, blk],
        out_specs=blk,
        out_shape=jax.ShapeDtypeStruct((N_NODES, DIM_H), _f32),
    )(a, b)


def _mesh():
    return plsc.VectorSubcoreMesh(core_axis_name="c", subcore_axis_name="s",
                                  num_cores=NC, num_subcores=NS)


_SC_PARAMS = pltpu.CompilerParams(needs_layout_passes=False,
                                  use_tc_tiling_on_sc=False)


def _pass_a_body(q_hbm, k_hbm, eb_hbm, src_hbm, dst_hbm, z8_hbm,
                 ex_hbm, den0_hbm, den1_hbm,
                 src_v, dst_v, qg, kg, bg, exb, den_sh, sem_g):
    c = lax.axis_index("c")
    s = lax.axis_index("s")
    wid = s * NC + c
    row0 = s * ROW_STRIDE
    pltpu.sync_copy(z8_hbm.at[pl.ds(row0, ROW_WIN)],
                    den_sh.at[pl.ds(row0, ROW_WIN)])
    plsc.subcore_barrier()

    base0 = wid * EDGES_PER_TILE

    def ebase(it):
        return base0 + it * CHUNK

    def load_i(it, u):
        pltpu.sync_copy(src_hbm.at[pl.ds(ebase(it), CHUNK)], src_v.at[u])
        pltpu.sync_copy(dst_hbm.at[pl.ds(ebase(it), CHUNK)], dst_v.at[u])

    def issue_g(it, u):
        pltpu.async_copy(q_hbm.at[dst_v.at[u]], qg.at[u], sem_g.at[u])
        pltpu.async_copy(k_hbm.at[src_v.at[u]], kg.at[u], sem_g.at[u])
        pltpu.async_copy(eb_hbm.at[pl.ds(ebase(it), CHUNK)], bg.at[u],
                         sem_g.at[u])

    def wait_g(u):
        pltpu.make_async_copy(q_hbm.at[dst_v.at[u]], qg.at[u],
                              sem_g.at[u]).wait()
        pltpu.make_async_copy(k_hbm.at[src_v.at[u]], kg.at[u],
                              sem_g.at[u]).wait()
        pltpu.make_async_copy(eb_hbm.at[pl.ds(0, CHUNK)], bg.at[u],
                              sem_g.at[u]).wait()

    def store_o(it, u):
        pltpu.sync_copy(exb.at[u], ex_hbm.at[pl.ds(ebase(it), CHUNK)])
        pltpu.sync_copy(exb.at[u], den_sh.at[dst_v.at[u]], add=True)

    def compute(u):
        qr = qg.at[u]
        kr = kg.at[u]
        br = bg.at[u]
        er = exb.at[u]

        def gbody(g, carry):
            row = g * 16 + _iota16()
            for h in range(N_HEADS):
                acc = jnp.zeros((16,), _f32)
                for d in range(HEAD_DIM):
                    col = _full16(h * HEAD_DIM + d)
                    acc = acc + (plsc.load_gather(qr, [row, col]) *
                                 plsc.load_gather(kr, [row, col]))
                bias = plsc.load_gather(br, [row, _full16(h)])
                ex = jnp.exp(acc * INV_SQRT_D + bias)
                plsc.store_scatter(er, [row, _full16(h)], ex)
            return carry

        lax.fori_loop(0, CHUNK // 16, gbody, 0)

    load_i(0, 0)
    issue_g(0, 0)

    def pair(j, carry):
        for u in range(2):
            it = j * 2 + u
            load_i(it + 1, u ^ 1)
            issue_g(it + 1, u ^ 1)
            wait_g(u)
            compute(u)
            store_o(it, u)
        return carry

    lax.fori_loop(0, (N_CHUNKS - 1) // 2, pair, 0)

    wait_g(0)
    compute(0)
    store_o(N_CHUNKS - 1, 0)

    plsc.subcore_barrier()

    @pl.when(c == 0)
    def _():
        pltpu.sync_copy(den_sh.at[pl.ds(row0, ROW_WIN)],
                        den0_hbm.at[pl.ds(row0, ROW_WIN)])

    @pl.when(c == 1)
    def _():
        pltpu.sync_copy(den_sh.at[pl.ds(row0, ROW_WIN)],
                        den1_hbm.at[pl.ds(row0, ROW_WIN)])


def _pass_a(Q, K, edge_bias, src, dst, z8):
    out = (jax.ShapeDtypeStruct((N_EDGES, N_HEADS), _f32),
           jax.ShapeDtypeStruct((N_NODES, N_HEADS), _f32),
           jax.ShapeDtypeStruct((N_NODES, N_HEADS), _f32))
    k = pl.kernel(
        _pass_a_body,
        out_type=out,
        mesh=_mesh(),
        compiler_params=_SC_PARAMS,
        scratch_types=[
            pltpu.VMEM((2, CHUNK), _i32),
            pltpu.VMEM((2, CHUNK), _i32),
            pltpu.VMEM((2, CHUNK, DIM_H), _f32),
            pltpu.VMEM((2, CHUNK, DIM_H), _f32),
            pltpu.VMEM((2, CHUNK, N_HEADS), _f32),
            pltpu.VMEM((2, CHUNK, N_HEADS), _f32),
            pltpu.VMEM_SHARED((N_NODES, N_HEADS), _f32),
            pltpu.SemaphoreType.DMA((2,)),
        ],
    )
    return k(Q, K, edge_bias, src, dst, z8)


def _pass_b_body(v_hbm, ex_hbm, den0_hbm, den1_hbm, src_hbm, dst_hbm, z128_hbm,
                 alpha_hbm, agg0_hbm, agg1_hbm,
                 src_v, dst_v, vg, exb, d0g, d1g, agg_sh, sem_g):
    c = lax.axis_index("c")
    s = lax.axis_index("s")
    wid = s * NC + c
    row0 = s * ROW_STRIDE
    pltpu.sync_copy(z128_hbm.at[pl.ds(row0, ROW_WIN)],
                    agg_sh.at[pl.ds(row0, ROW_WIN)])
    plsc.subcore_barrier()

    base0 = wid * EDGES_PER_TILE

    def ebase(it):
        return base0 + it * CHUNK

    def load_i(it, u):
        pltpu.sync_copy(src_hbm.at[pl.ds(ebase(it), CHUNK)], src_v.at[u])
        pltpu.sync_copy(dst_hbm.at[pl.ds(ebase(it), CHUNK)], dst_v.at[u])

    def issue_g(it, u):
        pltpu.async_copy(ex_hbm.at[pl.ds(ebase(it), CHUNK)], exb.at[u],
                         sem_g.at[u])
        pltpu.async_copy(den0_hbm.at[dst_v.at[u]], d0g.at[u], sem_g.at[u])
        pltpu.async_copy(den1_hbm.at[dst_v.at[u]], d1g.at[u], sem_g.at[u])
        pltpu.async_copy(v_hbm.at[src_v.at[u]], vg.at[u], sem_g.at[u])

    def wait_g(u):
        pltpu.make_async_copy(ex_hbm.at[pl.ds(0, CHUNK)], exb.at[u],
                              sem_g.at[u]).wait()
        pltpu.make_async_copy(den0_hbm.at[dst_v.at[u]], d0g.at[u],
                              sem_g.at[u]).wait()
        pltpu.make_async_copy(den1_hbm.at[dst_v.at[u]], d1g.at[u],
                              sem_g.at[u]).wait()
        pltpu.make_async_copy(v_hbm.at[src_v.at[u]], vg.at[u],
                              sem_g.at[u]).wait()

    def store_o(it, u):
        pltpu.sync_copy(exb.at[u], alpha_hbm.at[pl.ds(ebase(it), CHUNK)])
        pltpu.sync_copy(vg.at[u], agg_sh.at[dst_v.at[u]], add=True)

    def compute(u):
        vr = vg.at[u]
        er = exb.at[u]
        d0r = d0g.at[u]
        d1r = d1g.at[u]
        ar = er

        def gbody(g, carry):
            row = g * 16 + _iota16()
            for h in range(N_HEADS):
                fh = _full16(h)
                ex = plsc.load_gather(er, [row, fh])
                den = (plsc.load_gather(d0r, [row, fh]) +
                       plsc.load_gather(d1r, [row, fh]))
                al = ex / (den + 1e-16)
                plsc.store_scatter(ar, [row, fh], al)
                for d in range(HEAD_DIM):
                    col = _full16(h * HEAD_DIM + d)
                    vv = plsc.load_gather(vr, [row, col])
                    plsc.store_scatter(vr, [row, col], vv * al)
            return carry

        lax.fori_loop(0, CHUNK // 16, gbody, 0)

    load_i(0, 0)
    issue_g(0, 0)

    def pair(j, carry):
        for u in range(2):
            it = j * 2 + u
            load_i(it + 1, u ^ 1)
            issue_g(it + 1, u ^ 1)
            wait_g(u)
            compute(u)
            store_o(it, u)
        return carry

    lax.fori_loop(0, (N_CHUNKS - 1) // 2, pair, 0)

    wait_g(0)
    compute(0)
    store_o(N_CHUNKS - 1, 0)

    plsc.subcore_barrier()

    @pl.when(c == 0)
    def _():
        pltpu.sync_copy(agg_sh.at[pl.ds(row0, ROW_WIN)],
                        agg0_hbm.at[pl.ds(row0, ROW_WIN)])

    @pl.when(c == 1)
    def _():
        pltpu.sync_copy(agg_sh.at[pl.ds(row0, ROW_WIN)],
                        agg1_hbm.at[pl.ds(row0, ROW_WIN)])


def _pass_b(V, ex, den0, den1, src, dst, z128):
    out = (jax.ShapeDtypeStruct((N_EDGES, N_HEADS), _f32),
           jax.ShapeDtypeStruct((N_NODES, DIM_H), _f32),
           jax.ShapeDtypeStruct((N_NODES, DIM_H), _f32))
    k = pl.kernel(
        _pass_b_body,
        out_type=out,
        mesh=_mesh(),
        compiler_params=_SC_PARAMS,
        scratch_types=[
            pltpu.VMEM((2, CHUNK), _i32),
            pltpu.VMEM((2, CHUNK), _i32),
            pltpu.VMEM((2, CHUNK, DIM_H), _f32),
            pltpu.VMEM((2, CHUNK, N_HEADS), _f32),
            pltpu.VMEM((2, CHUNK, N_HEADS), _f32),
            pltpu.VMEM((2, CHUNK, N_HEADS), _f32),
            pltpu.VMEM_SHARED((N_NODES, DIM_H), _f32),
            pltpu.SemaphoreType.DMA((2,)),
        ],
    )
    return k(V, ex, den0, den1, src, dst, z128)


def kernel(x, edge_index, edge_bias, WQ, bQ, WK, bK, WV, bV):
    src = edge_index[0]
    dst = edge_index[1]
    z8 = jnp.zeros((N_NODES, N_HEADS), _f32)
    z128 = jnp.zeros((N_NODES, DIM_H), _f32)
    Q, K, V = _qkv(x, WQ, WK, WV, bQ, bK, bV)
    ex, den0, den1 = _pass_a(Q, K, edge_bias, src, dst, z8)
    alpha, agg0, agg1 = _pass_b(V, ex, den0, den1, src, dst, z128)
    h = _combine(agg0, agg1)
    return (h, alpha)


# speedup vs baseline: 12.9582x; 1.0608x over previous
import jax
import jax.numpy as jnp
from jax import lax
from jax.experimental import pallas as pl
from jax.experimental.pallas import tpu as pltpu
from jax.experimental.pallas import tpu_sc as plsc

N_NODES = 10000
N_EDGES = 320000
DIM_H = 128
N_HEADS = 8
HEAD_DIM = 16
INV_SQRT_D = 0.25

NC = 2
NS = 16
NW = NC * NS
EDGES_PER_TILE = N_EDGES // NW
CHUNK = 80
N_CHUNKS = EDGES_PER_TILE // CHUNK
N_GROUPS = CHUNK // 16
ROW_STRIDE = 624
ROW_WIN = 640

_f32 = jnp.float32
_i32 = jnp.int32


def _iota16():
    return lax.iota(_i32, 16)


def _full16(v):
    return jnp.full((16,), v, _i32)


_QKV_BLOCK = 1000
_DN = (((1,), (1,)), ((), ()))


def _qkv_body(x_ref, wq_ref, wk_ref, wv_ref, bq_ref, bk_ref, bv_ref,
              q_ref, k_ref, v_ref):
    xx = x_ref[...]
    q_ref[...] = lax.dot_general(xx, wq_ref[...], _DN,
                                 preferred_element_type=_f32) + bq_ref[...]
    k_ref[...] = lax.dot_general(xx, wk_ref[...], _DN,
                                 preferred_element_type=_f32) + bk_ref[...]
    v_ref[...] = lax.dot_general(xx, wv_ref[...], _DN,
                                 preferred_element_type=_f32) + bv_ref[...]


def _qkv(x, WQ, WK, WV, bQ, bK, bV):
    n_blk = N_NODES // _QKV_BLOCK
    blk = pl.BlockSpec((_QKV_BLOCK, DIM_H), lambda i: (i, 0))
    wblk = pl.BlockSpec((DIM_H, DIM_H), lambda i: (0, 0))
    bblk = pl.BlockSpec((1, DIM_H), lambda i: (0, 0))
    out = jax.ShapeDtypeStruct((N_NODES, DIM_H), _f32)
    return pl.pallas_call(
        _qkv_body,
        grid=(n_blk,),
        in_specs=[blk, wblk, wblk, wblk, bblk, bblk, bblk],
        out_specs=[blk, blk, blk],
        out_shape=[out, out, out],
    )(x, WQ, WK, WV, bQ.reshape(1, DIM_H), bK.reshape(1, DIM_H),
      bV.reshape(1, DIM_H))


def _combine_body(a_ref, b_ref, o_ref):
    o_ref[...] = a_ref[...] + b_ref[...]


def _combine(a, b):
    n_blk = N_NODES // _QKV_BLOCK
    d = a.shape[1]
    blk = pl.BlockSpec((_QKV_BLOCK, d), lambda i: (i, 0))
    return pl.pallas_call(
        _combine_body,
        grid=(n_blk,),
        in_specs=[blk, blk],
        out_specs=blk,
        out_shape=jax.ShapeDtypeStruct((N_NODES, d), _f32),
    )(a, b)


def _mesh():
    return plsc.VectorSubcoreMesh(core_axis_name="c", subcore_axis_name="s",
                                  num_cores=NC, num_subcores=NS)


_SC_PARAMS = pltpu.CompilerParams(needs_layout_passes=False,
                                  use_tc_tiling_on_sc=False)


def _fill_sc(big_src, big_dst, sc_src, sc_dst, it):
    off = it * CHUNK
    for g in range(N_GROUPS):
        lanes = off + g * 16 + _iota16()
        sv = plsc.load_gather(big_src, [lanes])
        dv = plsc.load_gather(big_dst, [lanes])
        sc_src[pl.ds(g * 16, 16)] = sv
        sc_dst[pl.ds(g * 16, 16)] = dv


def _pass_a_body(q_hbm, k_hbm, eb_hbm, src_hbm, dst_hbm, z8_hbm,
                 ex_hbm, den0_hbm, den1_hbm,
                 src_big, dst_big, src_sc, dst_sc, qg, kg, bg, exb, den_sh,
                 sem_g, sem_o):
    c = lax.axis_index("c")
    s = lax.axis_index("s")
    wid = s * NC + c
    row0 = s * ROW_STRIDE
    base0 = wid * EDGES_PER_TILE

    pltpu.sync_copy(src_hbm.at[pl.ds(base0, EDGES_PER_TILE)], src_big)
    pltpu.sync_copy(dst_hbm.at[pl.ds(base0, EDGES_PER_TILE)], dst_big)
    pltpu.sync_copy(z8_hbm.at[pl.ds(row0, ROW_WIN)],
                    den_sh.at[pl.ds(row0, ROW_WIN)])
    plsc.subcore_barrier()

    def ebase(it):
        return base0 + it * CHUNK

    def issue_g(it, u):
        pltpu.async_copy(q_hbm.at[dst_sc.at[u]], qg.at[u], sem_g.at[u])
        pltpu.async_copy(k_hbm.at[src_sc.at[u]], kg.at[u], sem_g.at[u])
        pltpu.async_copy(eb_hbm.at[pl.ds(ebase(it), CHUNK)], bg.at[u],
                         sem_g.at[u])

    def wait_g(u):
        pltpu.make_async_copy(q_hbm.at[dst_sc.at[u]], qg.at[u],
                              sem_g.at[u]).wait()
        pltpu.make_async_copy(k_hbm.at[src_sc.at[u]], kg.at[u],
                              sem_g.at[u]).wait()
        pltpu.make_async_copy(eb_hbm.at[pl.ds(0, CHUNK)], bg.at[u],
                              sem_g.at[u]).wait()

    def store_o(it, u):
        pltpu.sync_copy(exb.at[u], den_sh.at[dst_sc.at[u]], add=True)
        pltpu.async_copy(exb.at[u], ex_hbm.at[pl.ds(ebase(it), CHUNK)],
                         sem_o.at[u])

    def wait_o(u):
        pltpu.make_async_copy(exb.at[u], ex_hbm.at[pl.ds(0, CHUNK)],
                              sem_o.at[u]).wait()

    def compute(u):
        qr = qg.at[u]
        kr = kg.at[u]
        br = bg.at[u]
        er = exb.at[u]

        def gbody(g, carry):
            row = g * 16 + _iota16()
            for h in range(N_HEADS):
                acc = jnp.zeros((16,), _f32)
                for d in range(HEAD_DIM):
                    col = _full16(h * HEAD_DIM + d)
                    acc = acc + (plsc.load_gather(qr, [row, col]) *
                                 plsc.load_gather(kr, [row, col]))
                bias = plsc.load_gather(br, [row, _full16(h)])
                ex = jnp.exp(acc * INV_SQRT_D + bias)
                plsc.store_scatter(er, [row, _full16(h)], ex)
            return carry

        lax.fori_loop(0, N_GROUPS, gbody, 0)

    _fill_sc(src_big, dst_big, src_sc.at[0], dst_sc.at[0], 0)
    issue_g(0, 0)

    def pair(j, carry):
        for u in range(2):
            it = j * 2 + u
            _fill_sc(src_big, dst_big, src_sc.at[u ^ 1], dst_sc.at[u ^ 1],
                     it + 1)
            issue_g(it + 1, u ^ 1)
            wait_g(u)

            @pl.when(it >= 2)
            def _():
                wait_o(u)

            compute(u)
            store_o(it, u)
        return carry

    lax.fori_loop(0, (N_CHUNKS - 1) // 2, pair, 0)

    wait_g(0)
    wait_o(0)
    compute(0)
    store_o(N_CHUNKS - 1, 0)
    wait_o(1)
    wait_o(0)

    plsc.subcore_barrier()

    @pl.when(c == 0)
    def _():
        pltpu.sync_copy(den_sh.at[pl.ds(row0, ROW_WIN)],
                        den0_hbm.at[pl.ds(row0, ROW_WIN)])

    @pl.when(c == 1)
    def _():
        pltpu.sync_copy(den_sh.at[pl.ds(row0, ROW_WIN)],
                        den1_hbm.at[pl.ds(row0, ROW_WIN)])


def _pass_a(Q, K, edge_bias, src, dst, z8):
    out = (jax.ShapeDtypeStruct((N_EDGES, N_HEADS), _f32),
           jax.ShapeDtypeStruct((N_NODES, N_HEADS), _f32),
           jax.ShapeDtypeStruct((N_NODES, N_HEADS), _f32))
    k = pl.kernel(
        _pass_a_body,
        out_type=out,
        mesh=_mesh(),
        compiler_params=_SC_PARAMS,
        scratch_types=[
            pltpu.VMEM((EDGES_PER_TILE,), _i32),
            pltpu.VMEM((EDGES_PER_TILE,), _i32),
            pltpu.VMEM((2, CHUNK), _i32),
            pltpu.VMEM((2, CHUNK), _i32),
            pltpu.VMEM((2, CHUNK, DIM_H), _f32),
            pltpu.VMEM((2, CHUNK, DIM_H), _f32),
            pltpu.VMEM((2, CHUNK, N_HEADS), _f32),
            pltpu.VMEM((2, CHUNK, N_HEADS), _f32),
            pltpu.VMEM_SHARED((N_NODES, N_HEADS), _f32),
            pltpu.SemaphoreType.DMA((2,)),
            pltpu.SemaphoreType.DMA((2,)),
        ],
    )
    return k(Q, K, edge_bias, src, dst, z8)


def _pass_b_body(v_hbm, ex_hbm, den_hbm, src_hbm, dst_hbm, z128_hbm,
                 alpha_hbm, agg0_hbm, agg1_hbm,
                 src_big, dst_big, src_sc, dst_sc, vg, exb, dg, agg_sh,
                 sem_g, sem_o):
    c = lax.axis_index("c")
    s = lax.axis_index("s")
    wid = s * NC + c
    row0 = s * ROW_STRIDE
    base0 = wid * EDGES_PER_TILE

    pltpu.sync_copy(src_hbm.at[pl.ds(base0, EDGES_PER_TILE)], src_big)
    pltpu.sync_copy(dst_hbm.at[pl.ds(base0, EDGES_PER_TILE)], dst_big)
    pltpu.sync_copy(z128_hbm.at[pl.ds(row0, ROW_WIN)],
                    agg_sh.at[pl.ds(row0, ROW_WIN)])
    plsc.subcore_barrier()

    def ebase(it):
        return base0 + it * CHUNK

    def issue_g(it, u):
        pltpu.async_copy(ex_hbm.at[pl.ds(ebase(it), CHUNK)], exb.at[u],
                         sem_g.at[u])
        pltpu.async_copy(den_hbm.at[dst_sc.at[u]], dg.at[u], sem_g.at[u])
        pltpu.async_copy(v_hbm.at[src_sc.at[u]], vg.at[u], sem_g.at[u])

    def wait_g(u):
        pltpu.make_async_copy(ex_hbm.at[pl.ds(0, CHUNK)], exb.at[u],
                              sem_g.at[u]).wait()
        pltpu.make_async_copy(den_hbm.at[dst_sc.at[u]], dg.at[u],
                              sem_g.at[u]).wait()
        pltpu.make_async_copy(v_hbm.at[src_sc.at[u]], vg.at[u],
                              sem_g.at[u]).wait()

    def store_o(it, u):
        pltpu.sync_copy(vg.at[u], agg_sh.at[dst_sc.at[u]], add=True)
        pltpu.async_copy(exb.at[u], alpha_hbm.at[pl.ds(ebase(it), CHUNK)],
                         sem_o.at[u])

    def wait_o(u):
        pltpu.make_async_copy(exb.at[u], alpha_hbm.at[pl.ds(0, CHUNK)],
                              sem_o.at[u]).wait()

    def compute(u):
        vr = vg.at[u]
        er = exb.at[u]
        dr = dg.at[u]

        def gbody(g, carry):
            row = g * 16 + _iota16()
            for h in range(N_HEADS):
                fh = _full16(h)
                ex = plsc.load_gather(er, [row, fh])
                den = plsc.load_gather(dr, [row, fh])
                al = ex / (den + 1e-16)
                plsc.store_scatter(er, [row, fh], al)
                for d in range(HEAD_DIM):
                    col = _full16(h * HEAD_DIM + d)
                    vv = plsc.load_gather(vr, [row, col])
                    plsc.store_scatter(vr, [row, col], vv * al)
            return carry

        lax.fori_loop(0, N_GROUPS, gbody, 0)

    _fill_sc(src_big, dst_big, src_sc.at[0], dst_sc.at[0], 0)
    issue_g(0, 0)

    def pair(j, carry):
        for u in range(2):
            it = j * 2 + u
            _fill_sc(src_big, dst_big, src_sc.at[u ^ 1], dst_sc.at[u ^ 1],
                     it + 1)

            @pl.when(it >= 1)
            def _():
                wait_o(u ^ 1)

            issue_g(it + 1, u ^ 1)
            wait_g(u)
            compute(u)
            store_o(it, u)
        return carry

    lax.fori_loop(0, (N_CHUNKS - 1) // 2, pair, 0)

    wait_g(0)
    wait_o(1)
    compute(0)
    store_o(N_CHUNKS - 1, 0)
    wait_o(0)

    plsc.subcore_barrier()

    @pl.when(c == 0)
    def _():
        pltpu.sync_copy(agg_sh.at[pl.ds(row0, ROW_WIN)],
                        agg0_hbm.at[pl.ds(row0, ROW_WIN)])

    @pl.when(c == 1)
    def _():
        pltpu.sync_copy(agg_sh.at[pl.ds(row0, ROW_WIN)],
                        agg1_hbm.at[pl.ds(row0, ROW_WIN)])


def _pass_b(V, ex, den, src, dst, z128):
    out = (jax.ShapeDtypeStruct((N_EDGES, N_HEADS), _f32),
           jax.ShapeDtypeStruct((N_NODES, DIM_H), _f32),
           jax.ShapeDtypeStruct((N_NODES, DIM_H), _f32))
    k = pl.kernel(
        _pass_b_body,
        out_type=out,
        mesh=_mesh(),
        compiler_params=_SC_PARAMS,
        scratch_types=[
            pltpu.VMEM((EDGES_PER_TILE,), _i32),
            pltpu.VMEM((EDGES_PER_TILE,), _i32),
            pltpu.VMEM((2, CHUNK), _i32),
            pltpu.VMEM((2, CHUNK), _i32),
            pltpu.VMEM((2, CHUNK, DIM_H), _f32),
            pltpu.VMEM((2, CHUNK, N_HEADS), _f32),
            pltpu.VMEM((2, CHUNK, N_HEADS), _f32),
            pltpu.VMEM_SHARED((N_NODES, DIM_H), _f32),
            pltpu.SemaphoreType.DMA((2,)),
            pltpu.SemaphoreType.DMA((2,)),
        ],
    )
    return k(V, ex, den, src, dst, z128)


def kernel(x, edge_index, edge_bias, WQ, bQ, WK, bK, WV, bV):
    src = edge_index[0]
    dst = edge_index[1]
    z8 = jnp.zeros((N_NODES, N_HEADS), _f32)
    z128 = jnp.zeros((N_NODES, DIM_H), _f32)
    Q, K, V = _qkv(x, WQ, WK, WV, bQ, bK, bV)
    ex, den0, den1 = _pass_a(Q, K, edge_bias, src, dst, z8)
    den = _combine(den0, den1)
    alpha, agg0, agg1 = _pass_b(V, ex, den, src, dst, z128)
    h = _combine(agg0, agg1)
    return (h, alpha)
